# baseline (speedup 1.0000x reference)
import jax
import jax.numpy as jnp
from jax.experimental import pallas as pl
from jax.experimental.pallas import tpu as pltpu

_B = 4096
_P = 512
_E = 16
_H = 64
_V = 3

_BB = 512
_NB = _B // _BB


def _psig_kernel(idx_ref, emb_ref, w1_ref, b1_ref, w2_ref, b2_ref,
                 out_ref, c1_s, c2_s, c0sum_s, w2p_s):
    i = pl.program_id(0)

    @pl.when(i == 0)
    def _fold_weights():
        w1r = w1_ref[...].reshape(_P, _E, _H)
        emb = emb_ref[...]
        m = []
        for v in range(_V):
            row = emb[v:v + 1, :][:, :, None]
            m.append(jnp.sum(row * w1r, axis=1))
        c0 = m[0]
        c2 = (m[2] - 2.0 * m[1] + c0) * 0.5
        c1 = m[1] - c0 - c2
        c1h = c1.astype(jnp.bfloat16)
        c2h = c2.astype(jnp.bfloat16)
        c1l = (c1 - c1h.astype(jnp.float32)).astype(jnp.bfloat16)
        c2l = (c2 - c2h.astype(jnp.float32)).astype(jnp.bfloat16)
        c1_s[...] = jnp.concatenate([c1h, c1l], axis=1)
        c2_s[...] = jnp.concatenate([c2h, c2l], axis=1)
        c0sum_s[...] = (jnp.sum(c0, axis=0, keepdims=True)
                        + b1_ref[...].reshape(1, _H))
        w2 = w2_ref[...]
        w2h = w2.astype(jnp.bfloat16)
        w2l = (w2 - w2h.astype(jnp.float32)).astype(jnp.bfloat16)
        w2p_s[...] = jnp.concatenate([w2h, w2l], axis=1)

    xb = idx_ref[...].astype(jnp.float32).astype(jnp.bfloat16)
    x2 = xb * xb

    def mm(a, b_ref):
        return jax.lax.dot_general(
            a, b_ref[...], (((1,), (0,)), ((), ())),
            preferred_element_type=jnp.float32)

    r1 = mm(xb, c1_s)
    r2 = mm(x2, c2_s)
    pre = (r1[:, :_H] + r1[:, _H:] + r2[:, :_H] + r2[:, _H:]
           + c0sum_s[...])
    v1 = jnp.tanh(pre)
    v1h = v1.astype(jnp.bfloat16)
    v1l = (v1 - v1h.astype(jnp.float32)).astype(jnp.bfloat16)
    ra = mm(v1h, w2p_s)
    rb = mm(v1l, w2p_s)
    v2 = jnp.tanh(ra[:, :_H] + ra[:, _H:] + rb[:, :_H] + rb[:, _H:]
                  + b2_ref[...].reshape(1, _H))
    out_ref[...] = v2


def kernel(indices, emb_table, W1, b1, W2, b2):
    out = pl.pallas_call(
        _psig_kernel,
        grid=(_NB,),
        in_specs=[
            pl.BlockSpec((_BB, _P), lambda i: (i, 0)),
            pl.BlockSpec((_V, _E), lambda i: (0, 0)),
            pl.BlockSpec((_P * _E, _H), lambda i: (0, 0)),
            pl.BlockSpec((_H,), lambda i: (0,)),
            pl.BlockSpec((_H, _H), lambda i: (0, 0)),
            pl.BlockSpec((_H,), lambda i: (0,)),
        ],
        out_specs=pl.BlockSpec((_BB, _H), lambda i: (i, 0)),
        out_shape=jax.ShapeDtypeStruct((_B, _H), jnp.float32),
        scratch_shapes=[
            pltpu.VMEM((_P, 2 * _H), jnp.bfloat16),
            pltpu.VMEM((_P, 2 * _H), jnp.bfloat16),
            pltpu.VMEM((1, _H), jnp.float32),
            pltpu.VMEM((_H, 2 * _H), jnp.bfloat16),
        ],
        compiler_params=pltpu.CompilerParams(
            dimension_semantics=("arbitrary",),
        ),
    )(indices, emb_table, W1, b1, W2, b2)
    return out.reshape(-1)


# speedup vs baseline: 1.0828x; 1.0828x over previous
import jax
import jax.numpy as jnp
from jax.experimental import pallas as pl
from jax.experimental.pallas import tpu as pltpu

_B = 4096
_P = 512
_E = 16
_H = 64
_V = 3

_BB = 4096
_NB = _B // _BB


def _psig_kernel(idx_ref, emb_ref, w1_ref, b1_ref, w2_ref, b2_ref,
                 out_ref, c1_s, c2_s, c0sum_s, w2p_s):
    i = pl.program_id(0)

    @pl.when(i == 0)
    def _fold_weights():
        w1r = w1_ref[...].reshape(_P, _E, _H)
        emb = emb_ref[...]
        m = []
        for v in range(_V):
            row = emb[v:v + 1, :][:, :, None]
            m.append(jnp.sum(row * w1r, axis=1))
        c0 = m[0]
        c2 = (m[2] - 2.0 * m[1] + c0) * 0.5
        c1 = m[1] - c0 - c2
        c1h = c1.astype(jnp.bfloat16)
        c2h = c2.astype(jnp.bfloat16)
        c1l = (c1 - c1h.astype(jnp.float32)).astype(jnp.bfloat16)
        c2l = (c2 - c2h.astype(jnp.float32)).astype(jnp.bfloat16)
        c1_s[...] = jnp.concatenate([c1h, c1l], axis=1)
        c2_s[...] = jnp.concatenate([c2h, c2l], axis=1)
        c0sum_s[...] = (jnp.sum(c0, axis=0, keepdims=True)
                        + b1_ref[...].reshape(1, _H))
        w2 = w2_ref[...]
        w2h = w2.astype(jnp.bfloat16)
        w2l = (w2 - w2h.astype(jnp.float32)).astype(jnp.bfloat16)
        w2p_s[...] = jnp.concatenate([w2h, w2l], axis=1)

    xb = idx_ref[...].astype(jnp.float32).astype(jnp.bfloat16)
    x2 = xb * xb

    def mm(a, b_ref):
        return jax.lax.dot_general(
            a, b_ref[...], (((1,), (0,)), ((), ())),
            preferred_element_type=jnp.float32)

    r1 = mm(xb, c1_s)
    r2 = mm(x2, c2_s)
    pre = (r1[:, :_H] + r1[:, _H:] + r2[:, :_H] + r2[:, _H:]
           + c0sum_s[...])
    v1 = jnp.tanh(pre)
    v1h = v1.astype(jnp.bfloat16)
    v1l = (v1 - v1h.astype(jnp.float32)).astype(jnp.bfloat16)
    ra = mm(v1h, w2p_s)
    rb = mm(v1l, w2p_s)
    v2 = jnp.tanh(ra[:, :_H] + ra[:, _H:] + rb[:, :_H] + rb[:, _H:]
                  + b2_ref[...].reshape(1, _H))
    out_ref[...] = v2


def kernel(indices, emb_table, W1, b1, W2, b2):
    out = pl.pallas_call(
        _psig_kernel,
        grid=(_NB,),
        in_specs=[
            pl.BlockSpec((_BB, _P), lambda i: (i, 0)),
            pl.BlockSpec((_V, _E), lambda i: (0, 0)),
            pl.BlockSpec((_P * _E, _H), lambda i: (0, 0)),
            pl.BlockSpec((_H,), lambda i: (0,)),
            pl.BlockSpec((_H, _H), lambda i: (0, 0)),
            pl.BlockSpec((_H,), lambda i: (0,)),
        ],
        out_specs=pl.BlockSpec((_BB, _H), lambda i: (i, 0)),
        out_shape=jax.ShapeDtypeStruct((_B, _H), jnp.float32),
        scratch_shapes=[
            pltpu.VMEM((_P, 2 * _H), jnp.bfloat16),
            pltpu.VMEM((_P, 2 * _H), jnp.bfloat16),
            pltpu.VMEM((1, _H), jnp.float32),
            pltpu.VMEM((_H, 2 * _H), jnp.bfloat16),
        ],
        compiler_params=pltpu.CompilerParams(
            dimension_semantics=("arbitrary",),
        ),
    )(indices, emb_table, W1, b1, W2, b2)
    return out.reshape(-1)


# speedup vs baseline: 1.1664x; 1.0772x over previous
import jax
import jax.numpy as jnp
from jax.experimental import pallas as pl
from jax.experimental.pallas import tpu as pltpu

_B = 4096
_P = 512
_E = 16
_H = 64
_V = 3

_BB = 2048
_NB = _B // _BB


def _psig_kernel(idx_ref, emb_ref, w1_ref, b1_ref, w2_ref, b2_ref,
                 out_ref, c1_s, c2_s, c0sum_s, w2p_s):
    i = pl.program_id(0)

    @pl.when(i == 0)
    def _fold_weights():
        w1r = w1_ref[...].reshape(_P, _E, _H)
        emb = emb_ref[...]
        e0 = emb[0:1, :][:, :, None]
        e1 = emb[1:2, :][:, :, None]
        e2 = emb[2:3, :][:, :, None]
        g1 = 2.0 * e1 - 1.5 * e0 - 0.5 * e2
        g2 = 0.5 * (e0 + e2) - e1
        c1 = jnp.sum(g1 * w1r, axis=1)
        c2 = jnp.sum(g2 * w1r, axis=1)
        c1h = c1.astype(jnp.bfloat16)
        c2h = c2.astype(jnp.bfloat16)
        c1l = (c1 - c1h.astype(jnp.float32)).astype(jnp.bfloat16)
        c2l = (c2 - c2h.astype(jnp.float32)).astype(jnp.bfloat16)
        c1_s[...] = jnp.concatenate([c1h, c1l], axis=1)
        c2_s[...] = jnp.concatenate([c2h, c2l], axis=1)
        c0sum_s[...] = (jnp.sum(e0 * w1r, axis=(0, 1)).reshape(1, _H)
                        + b1_ref[...].reshape(1, _H))
        w2 = w2_ref[...]
        w2h = w2.astype(jnp.bfloat16)
        w2l = (w2 - w2h.astype(jnp.float32)).astype(jnp.bfloat16)
        w2p_s[...] = jnp.concatenate([w2h, w2l], axis=1)

    xb = idx_ref[...].astype(jnp.float32).astype(jnp.bfloat16)
    x2 = xb * xb

    def mm(a, b_ref):
        return jax.lax.dot_general(
            a, b_ref[...], (((1,), (0,)), ((), ())),
            preferred_element_type=jnp.float32)

    r1 = mm(xb, c1_s)
    r2 = mm(x2, c2_s)
    pre = (r1[:, :_H] + r1[:, _H:] + r2[:, :_H] + r2[:, _H:]
           + c0sum_s[...])
    v1 = jnp.tanh(pre)
    v1h = v1.astype(jnp.bfloat16)
    v1l = (v1 - v1h.astype(jnp.float32)).astype(jnp.bfloat16)
    ra = mm(v1h, w2p_s)
    rb = mm(v1l, w2p_s)
    v2 = jnp.tanh(ra[:, :_H] + ra[:, _H:] + rb[:, :_H] + rb[:, _H:]
                  + b2_ref[...].reshape(1, _H))
    out_ref[...] = v2


def kernel(indices, emb_table, W1, b1, W2, b2):
    out = pl.pallas_call(
        _psig_kernel,
        grid=(_NB,),
        in_specs=[
            pl.BlockSpec((_BB, _P), lambda i: (i, 0)),
            pl.BlockSpec((_V, _E), lambda i: (0, 0)),
            pl.BlockSpec((_P * _E, _H), lambda i: (0, 0)),
            pl.BlockSpec((_H,), lambda i: (0,)),
            pl.BlockSpec((_H, _H), lambda i: (0, 0)),
            pl.BlockSpec((_H,), lambda i: (0,)),
        ],
        out_specs=pl.BlockSpec((_BB, _H), lambda i: (i, 0)),
        out_shape=jax.ShapeDtypeStruct((_B, _H), jnp.float32),
        scratch_shapes=[
            pltpu.VMEM((_P, 2 * _H), jnp.bfloat16),
            pltpu.VMEM((_P, 2 * _H), jnp.bfloat16),
            pltpu.VMEM((1, _H), jnp.float32),
            pltpu.VMEM((_H, 2 * _H), jnp.bfloat16),
        ],
        compiler_params=pltpu.CompilerParams(
            dimension_semantics=("arbitrary",),
        ),
    )(indices, emb_table, W1, b1, W2, b2)
    return out.reshape(-1)
